# Initial kernel scaffold; baseline (speedup 1.0000x reference)
import jax
import jax.numpy as jnp
from jax.experimental import pallas as pl


def kernel(token_index, emb_table, fc_w, fc_b):
    raise NotImplementedError("write your pallas kernel here")


# speedup vs baseline: 93.4895x; 93.4895x over previous
import functools

import jax
import jax.numpy as jnp
from jax import lax
from jax.experimental import pallas as pl
from jax.experimental.pallas import tpu as pltpu
from jax.experimental.pallas import tpu_sc as plsc

VOCAB = 100000
EMB = 64
NUM_CLASS = 2
B = 16384
L = 200

NC = 2
NS = 16
NW = NC * NS
BAGS_PER_TEC = B // NW
GROUP = 16
NGROUPS = BAGS_PER_TEC // GROUP
POS_UNROLL = 8
PBLK = 12544
PGRID = (VOCAB + PBLK - 1) // PBLK


def _rne_bf16_hi(x_f32):
    u = lax.bitcast_convert_type(x_f32, jnp.int32)
    lsb = lax.shift_right_logical(u, 16) & 1
    r = u + jnp.int32(0x7FFF) + lsb
    return r & jnp.int32(-65536)


def _proj_body(emb_ref, w_ref, out_ref):
    pf = lax.dot_general(
        w_ref[...], emb_ref[...], (((1,), (1,)), ((), ())),
        preferred_element_type=jnp.float32)
    hi0 = _rne_bf16_hi(pf[0:1, :])
    hi1 = _rne_bf16_hi(pf[1:2, :])
    out_ref[...] = hi1 | lax.shift_right_logical(hi0, 16)


def _project(emb_table, fc_w):
    return pl.pallas_call(
        _proj_body,
        grid=(PGRID,),
        in_specs=[
            pl.BlockSpec((PBLK, EMB), lambda i: (i, 0)),
            pl.BlockSpec((NUM_CLASS, EMB), lambda i: (0, 0)),
        ],
        out_specs=pl.BlockSpec((1, PBLK), lambda i: (0, i)),
        out_shape=jax.ShapeDtypeStruct((1, VOCAB), jnp.int32),
    )(emb_table, fc_w)


def _bags_body(ptab_hbm, tok_hbm, bias_hbm, out_hbm,
               table_v, idx0_v, idx1_v, res_v, bias_v, sem0, sem1):
    c = lax.axis_index("c")
    s = lax.axis_index("s")
    wid = s * NC + c
    row0 = wid * BAGS_PER_TEC
    sems = (sem0, sem1)
    bufs = (idx0_v, idx1_v)
    tok0 = row0 * L

    pltpu.async_copy(tok_hbm.at[pl.ds(tok0, GROUP * L)], idx0_v, sem0)
    pltpu.async_copy(tok_hbm.at[pl.ds(tok0 + GROUP * L, GROUP * L)],
                     idx1_v, sem1)
    pltpu.sync_copy(ptab_hbm.at[0], table_v)
    pltpu.sync_copy(bias_hbm, bias_v)

    iota16 = lax.iota(jnp.int32, 16)
    lane_base = iota16 * L
    b0 = bias_v[0, :]
    b1 = bias_v[1, :]
    inv_l = jnp.float32(1.0 / L)
    zero = jnp.zeros((16,), jnp.float32)

    @pl.loop(0, NGROUPS, step=2)
    def _outer(g2):
        for bb in range(2):
            g = g2 + bb
            pltpu.make_async_copy(
                tok_hbm.at[pl.ds(0, GROUP * L)], bufs[bb], sems[bb]).wait()

            def pos_body(i, carry, bb=bb):
                a0, a1 = carry
                for k in range(POS_UNROLL):
                    p = i * POS_UNROLL + k
                    toks = plsc.load_gather(bufs[bb], [lane_base + p])
                    packed = plsc.load_gather(table_v, [toks])
                    a0 = a0 + plsc.bitcast(packed << 16, jnp.float32)
                    a1 = a1 + plsc.bitcast(packed & jnp.int32(-65536),
                                           jnp.float32)
                return a0, a1

            acc0, acc1 = lax.fori_loop(0, L // POS_UNROLL, pos_body,
                                       (zero, zero))
            r0 = acc0 * inv_l + b0
            r1 = acc1 * inv_l + b1
            base2 = (g * GROUP + iota16) * 2
            plsc.store_scatter(res_v, [base2], r0)
            plsc.store_scatter(res_v, [base2 + 1], r1)

            gg = g + 2

            @pl.when(gg < NGROUPS)
            def _(bb=bb, gg=gg):
                pltpu.async_copy(
                    tok_hbm.at[pl.ds(tok0 + gg * GROUP * L, GROUP * L)],
                    bufs[bb], sems[bb])

    pltpu.sync_copy(res_v, out_hbm.at[pl.ds(row0 * 2, BAGS_PER_TEC * 2)])


_bags = functools.partial(
    pl.kernel,
    out_type=jax.ShapeDtypeStruct((B * NUM_CLASS,), jnp.float32),
    mesh=plsc.VectorSubcoreMesh(
        core_axis_name="c", subcore_axis_name="s",
        num_cores=NC, num_subcores=NS),
    compiler_params=pltpu.CompilerParams(needs_layout_passes=False),
    scratch_types=[
        pltpu.VMEM((VOCAB,), jnp.int32),
        pltpu.VMEM((GROUP * L,), jnp.int32),
        pltpu.VMEM((GROUP * L,), jnp.int32),
        pltpu.VMEM((BAGS_PER_TEC * NUM_CLASS,), jnp.float32),
        pltpu.VMEM((NUM_CLASS, 16), jnp.float32),
        pltpu.SemaphoreType.DMA,
        pltpu.SemaphoreType.DMA,
    ],
)(_bags_body)


def kernel(token_index, emb_table, fc_w, fc_b):
    packed = _project(emb_table, fc_w)
    bias16 = jnp.broadcast_to(fc_b[:, None], (NUM_CLASS, 16))
    flat = _bags(packed, token_index.astype(jnp.int32).reshape(B * L), bias16)
    return flat.reshape(B, NUM_CLASS)
